# baseline (speedup 1.0000x reference)
import jax
import jax.numpy as jnp
from jax.experimental import pallas as pl
from jax.experimental.pallas import tpu as pltpu

_W = 256


def _copy_body(x_ref, o_ref):
    o_ref[...] = x_ref[...]


def _pallas_copy(x, rows_per_block):
    n, d = x.shape
    xw = x.reshape(n * d // _W, _W)
    nw = xw.shape[0]
    assert nw % rows_per_block == 0
    out = pl.pallas_call(
        _copy_body,
        grid=(nw // rows_per_block,),
        in_specs=[pl.BlockSpec((rows_per_block, _W), lambda i: (i, 0))],
        out_specs=pl.BlockSpec((rows_per_block, _W), lambda i: (i, 0)),
        out_shape=jax.ShapeDtypeStruct((nw, _W), x.dtype),
        compiler_params=pltpu.CompilerParams(
            dimension_semantics=("parallel",),
        ),
    )(xw)
    return out.reshape(n, d)


def kernel(user_emb, item_emb):
    u = _pallas_copy(user_emb, 5000)
    i = _pallas_copy(item_emb, 5000)
    return (u, i)


# speedup vs baseline: 1.3255x; 1.3255x over previous
import jax
import jax.numpy as jnp
from jax.experimental import pallas as pl
from jax.experimental.pallas import tpu as pltpu

_BM = 10000
_DEPTH = 4
_NBUF = 8
_D = 64


def _copy_body(u_ref, i_ref, uo_ref, io_ref, buf, in_sems, out_sems):
    chunks = []
    for k in range(u_ref.shape[0] // _BM):
        chunks.append((u_ref, uo_ref, k * _BM))
    for k in range(i_ref.shape[0] // _BM):
        chunks.append((i_ref, io_ref, k * _BM))
    n_chunks = len(chunks)

    def start_in(c):
        src, _, off = chunks[c]
        b = c % _NBUF
        pltpu.make_async_copy(
            src.at[pl.ds(off, _BM), :], buf.at[b], in_sems.at[b]
        ).start()

    def start_out(c):
        _, dst, off = chunks[c]
        b = c % _NBUF
        pltpu.make_async_copy(
            buf.at[b], dst.at[pl.ds(off, _BM), :], out_sems.at[b]
        ).start()

    def wait_in(c):
        src, _, off = chunks[c]
        b = c % _NBUF
        pltpu.make_async_copy(
            src.at[pl.ds(off, _BM), :], buf.at[b], in_sems.at[b]
        ).wait()

    def wait_out(c):
        _, dst, off = chunks[c]
        b = c % _NBUF
        pltpu.make_async_copy(
            buf.at[b], dst.at[pl.ds(off, _BM), :], out_sems.at[b]
        ).wait()

    for c in range(min(_DEPTH, n_chunks)):
        start_in(c)
    for c in range(n_chunks):
        wait_in(c)
        start_out(c)
        nxt = c + _DEPTH
        if nxt < n_chunks:
            if nxt >= _NBUF:
                wait_out(nxt - _NBUF)
            start_in(nxt)
    for c in range(max(0, n_chunks - _NBUF), n_chunks):
        wait_out(c)


def kernel(user_emb, item_emb):
    u, i = pl.pallas_call(
        _copy_body,
        in_specs=[
            pl.BlockSpec(memory_space=pl.ANY),
            pl.BlockSpec(memory_space=pl.ANY),
        ],
        out_specs=[
            pl.BlockSpec(memory_space=pl.ANY),
            pl.BlockSpec(memory_space=pl.ANY),
        ],
        out_shape=[
            jax.ShapeDtypeStruct(user_emb.shape, user_emb.dtype),
            jax.ShapeDtypeStruct(item_emb.shape, item_emb.dtype),
        ],
        scratch_shapes=[
            pltpu.VMEM((_NBUF, _BM, _D), jnp.float32),
            pltpu.SemaphoreType.DMA((_NBUF,)),
            pltpu.SemaphoreType.DMA((_NBUF,)),
        ],
    )(user_emb, item_emb)
    return (u, i)


# speedup vs baseline: 1.3262x; 1.0006x over previous
import jax
import jax.numpy as jnp
from jax.experimental import pallas as pl
from jax.experimental.pallas import tpu as pltpu

_W = 1024
_BM = 625
_DEPTH = 4
_NBUF = 8


def _copy_body(u_ref, i_ref, uo_ref, io_ref, buf, in_sems, out_sems):
    pairs = []
    for src, dst in ((u_ref, uo_ref), (i_ref, io_ref)):
        n, d = src.shape
        ws = src.reshape(n * d // _W, _W // d, d)
        wd = dst.reshape(n * d // _W, _W // d, d)
        for k in range(ws.shape[0] // _BM):
            pairs.append((ws, wd, k * _BM))
    n_chunks = len(pairs)

    def in_copy(c):
        src, _, off = pairs[c]
        b = c % _NBUF
        return pltpu.make_async_copy(
            src.at[pl.ds(off, _BM), :, :], buf.at[b], in_sems.at[b]
        )

    def out_copy(c):
        _, dst, off = pairs[c]
        b = c % _NBUF
        return pltpu.make_async_copy(
            buf.at[b], dst.at[pl.ds(off, _BM), :, :], out_sems.at[b]
        )

    for c in range(min(_DEPTH, n_chunks)):
        in_copy(c).start()
    for c in range(n_chunks):
        in_copy(c).wait()
        out_copy(c).start()
        nxt = c + _DEPTH
        if nxt < n_chunks:
            if nxt >= _NBUF:
                out_copy(nxt - _NBUF).wait()
            in_copy(nxt).start()
    for c in range(max(0, n_chunks - _NBUF), n_chunks):
        out_copy(c).wait()


def kernel(user_emb, item_emb):
    u, i = pl.pallas_call(
        _copy_body,
        in_specs=[
            pl.BlockSpec(memory_space=pl.ANY),
            pl.BlockSpec(memory_space=pl.ANY),
        ],
        out_specs=[
            pl.BlockSpec(memory_space=pl.ANY),
            pl.BlockSpec(memory_space=pl.ANY),
        ],
        out_shape=[
            jax.ShapeDtypeStruct(user_emb.shape, user_emb.dtype),
            jax.ShapeDtypeStruct(item_emb.shape, item_emb.dtype),
        ],
        scratch_shapes=[
            pltpu.VMEM((_NBUF, _BM, _W // 64, 64), jnp.float32),
            pltpu.SemaphoreType.DMA((_NBUF,)),
            pltpu.SemaphoreType.DMA((_NBUF,)),
        ],
    )(user_emb, item_emb)
    return (u, i)


# speedup vs baseline: 1.3272x; 1.0007x over previous
import jax
import jax.numpy as jnp
from jax.experimental import pallas as pl
from jax.experimental.pallas import tpu as pltpu

_NO = 125
_SB = 5
_DEPTH = 4
_NBUF = 8


def _copy_body(u_ref, i_ref, uo_ref, io_ref, buf, in_sems, out_sems):
    pairs = []
    for src, dst in ((u_ref, uo_ref), (i_ref, io_ref)):
        n, d = src.shape
        s = n * d // (_NO * 1024)
        ws = src.reshape(_NO, s, 16, 64)
        wd = dst.reshape(_NO, s, 16, 64)
        for k in range(s // _SB):
            pairs.append((ws, wd, k * _SB))
    n_chunks = len(pairs)

    def in_copy(c):
        src, _, off = pairs[c]
        b = c % _NBUF
        return pltpu.make_async_copy(
            src.at[:, pl.ds(off, _SB), :, :], buf.at[b], in_sems.at[b]
        )

    def out_copy(c):
        _, dst, off = pairs[c]
        b = c % _NBUF
        return pltpu.make_async_copy(
            buf.at[b], dst.at[:, pl.ds(off, _SB), :, :], out_sems.at[b]
        )

    for c in range(min(_DEPTH, n_chunks)):
        in_copy(c).start()
    for c in range(n_chunks):
        in_copy(c).wait()
        out_copy(c).start()
        nxt = c + _DEPTH
        if nxt < n_chunks:
            if nxt >= _NBUF:
                out_copy(nxt - _NBUF).wait()
            in_copy(nxt).start()
    for c in range(max(0, n_chunks - _NBUF), n_chunks):
        out_copy(c).wait()


def kernel(user_emb, item_emb):
    u, i = pl.pallas_call(
        _copy_body,
        in_specs=[
            pl.BlockSpec(memory_space=pl.ANY),
            pl.BlockSpec(memory_space=pl.ANY),
        ],
        out_specs=[
            pl.BlockSpec(memory_space=pl.ANY),
            pl.BlockSpec(memory_space=pl.ANY),
        ],
        out_shape=[
            jax.ShapeDtypeStruct(user_emb.shape, user_emb.dtype),
            jax.ShapeDtypeStruct(item_emb.shape, item_emb.dtype),
        ],
        scratch_shapes=[
            pltpu.VMEM((_NBUF, _NO, _SB, 16, 64), jnp.float32),
            pltpu.SemaphoreType.DMA((_NBUF,)),
            pltpu.SemaphoreType.DMA((_NBUF,)),
        ],
    )(user_emb, item_emb)
    return (u, i)
